# Initial kernel scaffold; baseline (speedup 1.0000x reference)
import jax
import jax.numpy as jnp
from jax.experimental import pallas as pl


def kernel(input, edge_index, edge_weight, W):
    raise NotImplementedError("write your pallas kernel here")


# speedup vs baseline: 2.2513x; 2.2513x over previous
import functools

import jax
import jax.numpy as jnp
from jax import lax
from jax.experimental import pallas as pl
from jax.experimental.pallas import tpu as pltpu
from jax.experimental.pallas import tpu_sc as plsc

N = 10000
D_IN = 256
D_OUT = 256
NC, NS = 2, 16
DH = D_OUT // NC
E = 160000
CHUNK = 128
PER_TILE = 10240
EP = PER_TILE * NS
NCHUNK = PER_TILE // CHUNK
NP = 10240
ROW_T = NP // NS
ROW_C = 128
LANES = 16


def _mm_body(x_ref, w_ref, o_ref):
    o_ref[...] = jnp.dot(x_ref[...], w_ref[...],
                         preferred_element_type=jnp.float32)


def _matmul_split(x, w):
    bm = 1000
    nm = N // bm
    return pl.pallas_call(
        _mm_body,
        grid=(NC, nm),
        in_specs=[
            pl.BlockSpec((bm, D_IN), lambda c, m: (m, 0)),
            pl.BlockSpec((D_IN, DH), lambda c, m: (0, c)),
        ],
        out_specs=pl.BlockSpec((bm, DH), lambda c, m: (c * nm + m, 0)),
        out_shape=jax.ShapeDtypeStruct((NC * N, DH), jnp.float32),
    )(x, w)


def _spmm_body(sup_hbm, src_hbm, dst_hbm, ew_hbm, out_hbm,
               sidx, didx, eww, rows, acc, sem):
    c = lax.axis_index("c")
    s = lax.axis_index("s")

    @pl.loop(0, ROW_C)
    def _zero(r):
        for k in range(DH // LANES):
            rows[r, pl.ds(k * LANES, LANES)] = jnp.zeros((LANES,),
                                                         jnp.float32)

    for j in range(ROW_T // ROW_C):
        pltpu.sync_copy(rows.at[pl.ds(0, ROW_C)],
                        acc.at[pl.ds(s * ROW_T + j * ROW_C, ROW_C)])
    plsc.subcore_barrier()

    coff = c * N

    @pl.loop(0, NCHUNK)
    def _edge_chunk(g):
        base = s * PER_TILE + g * CHUNK
        pltpu.sync_copy(src_hbm.at[pl.ds(base, CHUNK)], sidx)
        pltpu.sync_copy(dst_hbm.at[pl.ds(base, CHUNK)], didx)
        pltpu.sync_copy(ew_hbm.at[pl.ds(base, CHUNK)], eww)
        for j in range(CHUNK // LANES):
            sl = pl.ds(j * LANES, LANES)
            sidx[sl] = sidx[sl] + coff
        pltpu.async_copy(sup_hbm.at[sidx], rows, sem).wait()

        @pl.loop(0, CHUNK // LANES)
        def _scale(q):
            wv = eww[pl.ds(q * LANES, LANES)]
            for i in range(LANES):
                w = wv[i]
                e = q * LANES + i
                for k in range(DH // LANES):
                    sl = pl.ds(k * LANES, LANES)
                    rows[e, sl] = rows[e, sl] * w

        pltpu.sync_copy(rows, acc.at[didx], add=True)

    plsc.subcore_barrier()

    for j in range(ROW_T // ROW_C):
        r0 = s * ROW_T + j * ROW_C
        pltpu.sync_copy(acc.at[pl.ds(r0, ROW_C)], rows.at[pl.ds(0, ROW_C)])
        pltpu.sync_copy(rows.at[pl.ds(0, ROW_C)],
                        out_hbm.at[c, pl.ds(r0, ROW_C)])


def _spmm_sc(sup, src, dst, ew):
    mesh = plsc.VectorSubcoreMesh(core_axis_name="c", subcore_axis_name="s",
                                  num_cores=NC, num_subcores=NS)
    run = pl.kernel(
        _spmm_body,
        out_type=jax.ShapeDtypeStruct((NC, NP, DH), jnp.float32),
        mesh=mesh,
        scratch_types=[
            pltpu.VMEM((CHUNK,), jnp.int32),
            pltpu.VMEM((CHUNK,), jnp.int32),
            pltpu.VMEM((CHUNK,), jnp.float32),
            pltpu.VMEM((CHUNK, DH), jnp.float32),
            pltpu.VMEM_SHARED((NP, DH), jnp.float32),
            pltpu.SemaphoreType.DMA,
        ],
    )
    return run(sup, src, dst, ew)


def kernel(input, edge_index, edge_weight, W):
    src = edge_index[0].astype(jnp.int32)
    dst = edge_index[1].astype(jnp.int32)
    pad = EP - E
    src = jnp.pad(src, (0, pad))
    dst = jnp.pad(dst, (0, pad))
    ew = jnp.pad(edge_weight, (0, pad))
    sup = _matmul_split(input, W)
    out2 = _spmm_sc(sup, src, dst, ew)
    return out2[:, :N, :].transpose(1, 0, 2).reshape(N, D_OUT)


# speedup vs baseline: 3.3172x; 1.4735x over previous
import functools

import jax
import jax.numpy as jnp
from jax import lax
from jax.experimental import pallas as pl
from jax.experimental.pallas import tpu as pltpu
from jax.experimental.pallas import tpu_sc as plsc

N = 10000
D_IN = 256
D_OUT = 256
NC, NS = 2, 16
DH = D_OUT // NC
E = 160000
CHUNK = 64
PER_TILE = 10240
EP = PER_TILE * NS
NCHUNK = PER_TILE // CHUNK
NP = 10240
ROW_T = NP // NS
ROW_C = 64
LANES = 16
NBUF = 4


def _mm_body(x_ref, w_ref, o_ref):
    o_ref[...] = jnp.dot(x_ref[...], w_ref[...],
                         preferred_element_type=jnp.float32)


def _matmul_split(x, w):
    bm = 1000
    nm = N // bm
    return pl.pallas_call(
        _mm_body,
        grid=(NC, nm),
        in_specs=[
            pl.BlockSpec((bm, D_IN), lambda c, m: (m, 0)),
            pl.BlockSpec((D_IN, DH), lambda c, m: (0, c)),
        ],
        out_specs=pl.BlockSpec((bm, DH), lambda c, m: (c * nm + m, 0)),
        out_shape=jax.ShapeDtypeStruct((NC * N, DH), jnp.float32),
    )(x, w)


def _spmm_body(sup_hbm, meta_hbm, ew_hbm, out_hbm,
               meta, sidx, didx, eww, rows, acc,
               sm0, sm1, sm2, sm3, sg0, sg1, sg2, sg3, ss0, ss1, ss2, ss3,
               sw0, sw1, sw2, sw3):
    sm = (sm0, sm1, sm2, sm3)
    sg = (sg0, sg1, sg2, sg3)
    ss = (ss0, ss1, ss2, ss3)
    sw = (sw0, sw1, sw2, sw3)
    c = lax.axis_index("c")
    s = lax.axis_index("s")
    coff = c * N

    @pl.loop(0, ROW_C)
    def _zero(r):
        for k in range(DH // LANES):
            rows[0, r, pl.ds(k * LANES, LANES)] = jnp.zeros((LANES,),
                                                            jnp.float32)

    for j in range(ROW_T // ROW_C):
        pltpu.sync_copy(rows.at[0, pl.ds(0, ROW_C)],
                        acc.at[pl.ds(s * ROW_T + j * ROW_C, ROW_C)])
    plsc.subcore_barrier()

    def issue_meta(i, b):
        pltpu.async_copy(meta_hbm.at[s * NCHUNK + i], meta.at[b], sm[b])
        pltpu.async_copy(ew_hbm.at[s * NCHUNK + i], eww.at[b], sw[b])

    def wait_meta(b):
        pltpu.make_async_copy(meta_hbm.at[0], meta.at[b], sm[b]).wait()
        pltpu.make_async_copy(ew_hbm.at[0], eww.at[b], sw[b]).wait()

    def extract(b):
        for j in range(CHUNK // LANES):
            sl = pl.ds(j * LANES, LANES)
            sidx[b, sl] = meta[b, 0, sl] + coff
            didx[b, sl] = meta[b, 1, sl]

    def issue_gather(b):
        pltpu.async_copy(sup_hbm.at[sidx.at[b]], rows.at[b], sg[b])

    def wait_gather(b):
        pltpu.make_async_copy(sup_hbm.at[pl.ds(0, CHUNK)], rows.at[b],
                              sg[b]).wait()

    def scale(b):
        @pl.loop(0, CHUNK // LANES)
        def _sc(q):
            wv = eww[b, pl.ds(q * LANES, LANES)]
            for i in range(LANES):
                w = wv[i]
                e = q * LANES + i
                for k in range(DH // LANES):
                    sl = pl.ds(k * LANES, LANES)
                    rows[b, e, sl] = rows[b, e, sl] * w

    def issue_scatter(b):
        pltpu.async_copy(rows.at[b], acc.at[didx.at[b]], ss[b], add=True)

    def wait_scatter(b):
        pltpu.make_async_copy(sup_hbm.at[pl.ds(0, CHUNK)], rows.at[b],
                              ss[b]).wait()

    def step(i, ic, do_sw=True, do_nxt=True, do_meta=True):
        b = ic % NBUF
        bn = (ic + 1) % NBUF
        bn2 = (ic + 2) % NBUF
        if do_nxt:
            if do_sw:
                wait_scatter(bn)
            wait_meta(bn)
            extract(bn)
            issue_gather(bn)
        if do_meta:
            issue_meta(i + 2, bn2)
        wait_gather(b)
        scale(b)
        issue_scatter(b)

    issue_meta(0, 0)
    issue_meta(1, 1)
    wait_meta(0)
    extract(0)
    issue_gather(0)
    step(0, 0, do_sw=False)
    step(1, 1, do_sw=False)
    step(2, 2, do_sw=False)
    step(3, 3)

    @pl.loop(4, NCHUNK - 4, step=NBUF)
    def _main(i):
        for k in range(NBUF):
            step(i + k, k)

    step(NCHUNK - 4, 0)
    step(NCHUNK - 3, 1)
    step(NCHUNK - 2, 2, do_meta=False)
    step(NCHUNK - 1, 3, do_nxt=False, do_meta=False)
    for b in range(NBUF):
        wait_scatter(b)

    plsc.subcore_barrier()

    for j in range(ROW_T // ROW_C):
        r0 = s * ROW_T + j * ROW_C
        b = j % 2
        pltpu.sync_copy(acc.at[pl.ds(r0, ROW_C)], rows.at[b])
        pltpu.sync_copy(rows.at[b], out_hbm.at[c, pl.ds(r0, ROW_C)])


def _spmm_sc(sup, meta, ewc):
    mesh = plsc.VectorSubcoreMesh(core_axis_name="c", subcore_axis_name="s",
                                  num_cores=NC, num_subcores=NS)
    run = pl.kernel(
        _spmm_body,
        out_type=jax.ShapeDtypeStruct((NC, NP, DH), jnp.float32),
        mesh=mesh,
        scratch_types=[
            pltpu.VMEM((NBUF, 2, CHUNK), jnp.int32),
            pltpu.VMEM((NBUF, CHUNK), jnp.int32),
            pltpu.VMEM((NBUF, CHUNK), jnp.int32),
            pltpu.VMEM((NBUF, CHUNK), jnp.float32),
            pltpu.VMEM((NBUF, CHUNK, DH), jnp.float32),
            pltpu.VMEM_SHARED((NP, DH), jnp.float32),
        ] + [pltpu.SemaphoreType.DMA] * 16,
    )
    return run(sup, meta, ewc)


def kernel(input, edge_index, edge_weight, W):
    src = edge_index[0].astype(jnp.int32)
    dst = edge_index[1].astype(jnp.int32)
    pad = EP - E
    src = jnp.pad(src, (0, pad))
    dst = jnp.pad(dst, (0, pad))
    ew = jnp.pad(edge_weight, (0, pad))
    meta = jnp.stack([src.reshape(NS * NCHUNK, CHUNK),
                      dst.reshape(NS * NCHUNK, CHUNK)], axis=1)
    ewc = ew.reshape(NS * NCHUNK, CHUNK)
    sup = _matmul_split(input, W)
    out2 = _spmm_sc(sup, meta, ewc)
    return out2[:, :N, :].transpose(1, 0, 2).reshape(N, D_OUT)


# speedup vs baseline: 6.6806x; 2.0139x over previous
import functools

import jax
import jax.numpy as jnp
from jax import lax
from jax.experimental import pallas as pl
from jax.experimental.pallas import tpu as pltpu
from jax.experimental.pallas import tpu_sc as plsc

N = 10000
D_IN = 256
D_OUT = 256
NC, NS = 2, 16
DH = D_OUT // NC
E = 160000
CHUNK = 80
PER_TILE = E // NS
NCHUNK = PER_TILE // CHUNK
NP = 10240
ROW_T = NP // NS
CP = 80
LANES = 16
NBUF = 4


def _mm_body(x_ref, w_ref, o_ref):
    o_ref[...] = jnp.dot(x_ref[...], w_ref[...],
                         preferred_element_type=jnp.float32)


def _matmul_split(x, w):
    bm = 1000
    nm = N // bm
    return pl.pallas_call(
        _mm_body,
        grid=(NC, nm),
        in_specs=[
            pl.BlockSpec((bm, D_IN), lambda c, m: (m, 0)),
            pl.BlockSpec((D_IN, DH), lambda c, m: (0, c)),
        ],
        out_specs=pl.BlockSpec((bm, DH), lambda c, m: (c * nm + m, 0)),
        out_shape=jax.ShapeDtypeStruct((NC * N, DH), jnp.float32),
    )(x, w)


def _spmm_body(sup_hbm, src_hbm, dst_hbm, ew_hbm, out_hbm,
               sidx, didx, eww, rows, acc,
               sm0, sm1, sm2, sm3, sd0, sd1, sd2, sd3,
               sw0, sw1, sw2, sw3, sg0, sg1, sg2, sg3,
               ss0, ss1, ss2, ss3):
    sm = (sm0, sm1, sm2, sm3)
    sd = (sd0, sd1, sd2, sd3)
    sw = (sw0, sw1, sw2, sw3)
    sg = (sg0, sg1, sg2, sg3)
    ss = (ss0, ss1, ss2, ss3)
    c = lax.axis_index("c")
    s = lax.axis_index("s")
    coff = c * N

    @pl.loop(0, CP)
    def _zero(r):
        for k in range(DH // LANES):
            rows[0, r, pl.ds(k * LANES, LANES)] = jnp.zeros((LANES,),
                                                            jnp.float32)

    for j in range(ROW_T // CP):
        pltpu.sync_copy(rows.at[0],
                        acc.at[pl.ds(s * ROW_T + j * CP, CP)])
    plsc.subcore_barrier()

    def issue_meta(i, b):
        base = s * PER_TILE + i * CHUNK
        pltpu.async_copy(src_hbm.at[pl.ds(base, CHUNK)], sidx.at[b], sm[b])
        pltpu.async_copy(dst_hbm.at[pl.ds(base, CHUNK)], didx.at[b], sd[b])
        pltpu.async_copy(ew_hbm.at[pl.ds(base, CHUNK)], eww.at[b], sw[b])

    def wait_meta(b):
        pltpu.make_async_copy(src_hbm.at[pl.ds(0, CHUNK)], sidx.at[b],
                              sm[b]).wait()
        pltpu.make_async_copy(dst_hbm.at[pl.ds(0, CHUNK)], didx.at[b],
                              sd[b]).wait()
        pltpu.make_async_copy(ew_hbm.at[pl.ds(0, CHUNK)], eww.at[b],
                              sw[b]).wait()

    def extract(b):
        for j in range(CHUNK // LANES):
            sl = pl.ds(j * LANES, LANES)
            sidx[b, sl] = sidx[b, sl] + coff

    def issue_gather(b):
        pltpu.async_copy(sup_hbm.at[sidx.at[b]], rows.at[b], sg[b])

    def wait_gather(b):
        pltpu.make_async_copy(sup_hbm.at[pl.ds(0, CHUNK)], rows.at[b],
                              sg[b]).wait()

    def scale(b):
        @pl.loop(0, CHUNK // LANES)
        def _sc(q):
            wv = eww[b, pl.ds(q * LANES, LANES)]
            for i in range(LANES):
                w = wv[i]
                e = q * LANES + i
                for k in range(DH // LANES):
                    sl = pl.ds(k * LANES, LANES)
                    rows[b, e, sl] = rows[b, e, sl] * w

    def issue_scatter(b):
        pltpu.async_copy(rows.at[b], acc.at[didx.at[b]], ss[b], add=True)

    def wait_scatter(b):
        pltpu.make_async_copy(sup_hbm.at[pl.ds(0, CHUNK)], rows.at[b],
                              ss[b]).wait()

    def step(i, ic, do_sw=True, do_nxt=True, do_meta=True):
        b = ic % NBUF
        bn = (ic + 1) % NBUF
        bn2 = (ic + 2) % NBUF
        if do_nxt:
            wait_meta(bn)
            extract(bn)
            issue_gather(bn)
        if do_meta:
            if do_sw:
                wait_scatter(bn2)
            issue_meta(i + 2, bn2)
        wait_gather(b)
        scale(b)
        issue_scatter(b)

    issue_meta(0, 0)
    issue_meta(1, 1)
    wait_meta(0)
    extract(0)
    issue_gather(0)
    step(0, 0, do_sw=False)
    step(1, 1, do_sw=False)
    step(2, 2)
    step(3, 3)

    @pl.loop(4, NCHUNK - 5, step=NBUF)
    def _main(i):
        for k in range(NBUF):
            step(i + k, k)

    step(NCHUNK - 5, 0)
    step(NCHUNK - 4, 1)
    step(NCHUNK - 3, 2)
    step(NCHUNK - 2, 3, do_meta=False)
    step(NCHUNK - 1, 0, do_nxt=False, do_meta=False)
    for b in range(NBUF):
        wait_scatter(b)

    plsc.subcore_barrier()

    for j in range(ROW_T // CP):
        r0 = s * ROW_T + j * CP
        pltpu.sync_copy(acc.at[pl.ds(r0, CP)], rows.at[j % 2])
        pltpu.sync_copy(rows.at[j % 2], out_hbm.at[c, pl.ds(r0, CP)])


def _spmm_sc(sup, src, dst, ew):
    mesh = plsc.VectorSubcoreMesh(core_axis_name="c", subcore_axis_name="s",
                                  num_cores=NC, num_subcores=NS)
    run = pl.kernel(
        _spmm_body,
        out_type=jax.ShapeDtypeStruct((NC, NP, DH), jnp.float32),
        mesh=mesh,
        scratch_types=[
            pltpu.VMEM((NBUF, CHUNK), jnp.int32),
            pltpu.VMEM((NBUF, CHUNK), jnp.int32),
            pltpu.VMEM((NBUF, CHUNK), jnp.float32),
            pltpu.VMEM((NBUF, CHUNK, DH), jnp.float32),
            pltpu.VMEM_SHARED((NP, DH), jnp.float32),
        ] + [pltpu.SemaphoreType.DMA] * 20,
    )
    return run(sup, src, dst, ew)


def kernel(input, edge_index, edge_weight, W):
    ei = edge_index.astype(jnp.int32)
    sup = _matmul_split(input, W)
    out2 = _spmm_sc(sup, ei[0], ei[1], edge_weight)
    return out2[:, :N, :].transpose(1, 0, 2).reshape(N, D_OUT)


# speedup vs baseline: 8.1153x; 1.2148x over previous
import functools

import jax
import jax.numpy as jnp
from jax import lax
from jax.experimental import pallas as pl
from jax.experimental.pallas import tpu as pltpu
from jax.experimental.pallas import tpu_sc as plsc

N = 10000
D_IN = 256
D_OUT = 256
NC, NS = 2, 16
DH = D_OUT // NC
E = 160000
CHUNK = 80
PER_TILE = E // NS
NCHUNK = PER_TILE // CHUNK
NP = 10240
ROW_T = NP // NS
CP = 80
LANES = 16
NBUF = 4


def _mm_body(x_ref, w_ref, o_ref):
    o_ref[...] = jnp.dot(x_ref[...], w_ref[...],
                         preferred_element_type=jnp.float32)


def _matmul_split(x, w):
    bm = 1000
    nm = N // bm
    return pl.pallas_call(
        _mm_body,
        grid=(NC, nm),
        in_specs=[
            pl.BlockSpec((bm, D_IN), lambda c, m: (m, 0)),
            pl.BlockSpec((D_IN, DH), lambda c, m: (0, c)),
        ],
        out_specs=pl.BlockSpec((bm, DH), lambda c, m: (c * nm + m, 0)),
        out_shape=jax.ShapeDtypeStruct((NC * N, DH), jnp.float32),
    )(x, w)


def _spmm_body(sup_hbm, ei_hbm, ew_hbm, out_hbm,
               sidx, didx, eww, rows, acc,
               sm0, sm1, sm2, sm3, sd0, sd1, sd2, sd3,
               sw0, sw1, sw2, sw3, sg0, sg1, sg2, sg3,
               ss0, ss1, ss2, ss3):
    sm = (sm0, sm1, sm2, sm3)
    sd = (sd0, sd1, sd2, sd3)
    sw = (sw0, sw1, sw2, sw3)
    sg = (sg0, sg1, sg2, sg3)
    ss = (ss0, ss1, ss2, ss3)
    c = lax.axis_index("c")
    s = lax.axis_index("s")
    coff = c * N

    @pl.loop(0, CP)
    def _zero(r):
        for k in range(DH // LANES):
            rows[0, r, pl.ds(k * LANES, LANES)] = jnp.zeros((LANES,),
                                                            jnp.float32)

    for j in range(ROW_T // CP):
        pltpu.sync_copy(rows.at[0],
                        acc.at[pl.ds(s * ROW_T + j * CP, CP)])
    plsc.subcore_barrier()

    def issue_meta(i, b):
        base = s * PER_TILE + i * CHUNK
        pltpu.async_copy(ei_hbm.at[pl.ds(base, CHUNK)], sidx.at[b], sm[b])
        pltpu.async_copy(ei_hbm.at[pl.ds(E + base, CHUNK)], didx.at[b],
                         sd[b])
        pltpu.async_copy(ew_hbm.at[pl.ds(base, CHUNK)], eww.at[b], sw[b])

    def wait_meta(b):
        pltpu.make_async_copy(ei_hbm.at[pl.ds(0, CHUNK)], sidx.at[b],
                              sm[b]).wait()
        pltpu.make_async_copy(ei_hbm.at[pl.ds(0, CHUNK)], didx.at[b],
                              sd[b]).wait()
        pltpu.make_async_copy(ew_hbm.at[pl.ds(0, CHUNK)], eww.at[b],
                              sw[b]).wait()

    def extract(b):
        for j in range(CHUNK // LANES):
            sl = pl.ds(j * LANES, LANES)
            sidx[b, sl] = sidx[b, sl] + coff

    def issue_gather(b):
        pltpu.async_copy(sup_hbm.at[sidx.at[b]], rows.at[b], sg[b])

    def wait_gather(b):
        pltpu.make_async_copy(sup_hbm.at[pl.ds(0, CHUNK)], rows.at[b],
                              sg[b]).wait()

    def scale(b):
        @pl.loop(0, CHUNK // LANES)
        def _sc(q):
            wv = eww[b, pl.ds(q * LANES, LANES)]
            for i in range(LANES):
                w = wv[i]
                e = q * LANES + i
                for k in range(DH // LANES):
                    sl = pl.ds(k * LANES, LANES)
                    rows[b, e, sl] = rows[b, e, sl] * w

    def issue_scatter(b):
        pltpu.async_copy(rows.at[b], acc.at[didx.at[b]], ss[b], add=True)

    def wait_scatter(b):
        pltpu.make_async_copy(sup_hbm.at[pl.ds(0, CHUNK)], rows.at[b],
                              ss[b]).wait()

    def step(i, ic, do_sw=True, do_nxt=True, do_meta=True):
        b = ic % NBUF
        bn = (ic + 1) % NBUF
        bn2 = (ic + 2) % NBUF
        if do_nxt:
            wait_meta(bn)
            extract(bn)
            issue_gather(bn)
        if do_meta:
            if do_sw:
                wait_scatter(bn2)
            issue_meta(i + 2, bn2)
        wait_gather(b)
        scale(b)
        issue_scatter(b)

    issue_meta(0, 0)
    issue_meta(1, 1)
    wait_meta(0)
    extract(0)
    issue_gather(0)
    step(0, 0, do_sw=False)
    step(1, 1, do_sw=False)
    step(2, 2)
    step(3, 3)

    @pl.loop(4, NCHUNK - 5, step=NBUF)
    def _main(i):
        for k in range(NBUF):
            step(i + k, k)

    step(NCHUNK - 5, 0)
    step(NCHUNK - 4, 1)
    step(NCHUNK - 3, 2)
    step(NCHUNK - 2, 3, do_meta=False)
    step(NCHUNK - 1, 0, do_nxt=False, do_meta=False)
    for b in range(NBUF):
        wait_scatter(b)

    plsc.subcore_barrier()

    nout = jnp.where(s == NS - 1, (N - (NS - 1) * ROW_T) // CP, ROW_T // CP)
    col0 = pl.multiple_of(c * DH, 128)

    @pl.loop(0, nout)
    def _out(j):
        r0 = pl.multiple_of(s * ROW_T + j * CP, 8)
        pltpu.sync_copy(acc.at[pl.ds(r0, CP)], rows.at[0])
        pltpu.sync_copy(rows.at[0],
                        out_hbm.at[pl.ds(r0, CP), pl.ds(col0, DH)])


def _spmm_sc(sup, ei, ew):
    mesh = plsc.VectorSubcoreMesh(core_axis_name="c", subcore_axis_name="s",
                                  num_cores=NC, num_subcores=NS)
    run = pl.kernel(
        _spmm_body,
        out_type=jax.ShapeDtypeStruct((N, D_OUT), jnp.float32),
        mesh=mesh,
        scratch_types=[
            pltpu.VMEM((NBUF, CHUNK), jnp.int32),
            pltpu.VMEM((NBUF, CHUNK), jnp.int32),
            pltpu.VMEM((NBUF, CHUNK), jnp.float32),
            pltpu.VMEM((NBUF, CHUNK, DH), jnp.float32),
            pltpu.VMEM_SHARED((NP, DH), jnp.float32),
        ] + [pltpu.SemaphoreType.DMA] * 20,
    )
    return run(sup, ei, ew)


def kernel(input, edge_index, edge_weight, W):
    ei = edge_index.astype(jnp.int32).reshape(2 * E)
    sup = _matmul_split(input, W)
    return _spmm_sc(sup, ei, edge_weight)


# speedup vs baseline: 8.1501x; 1.0043x over previous
import functools

import jax
import jax.numpy as jnp
from jax import lax
from jax.experimental import pallas as pl
from jax.experimental.pallas import tpu as pltpu
from jax.experimental.pallas import tpu_sc as plsc

N = 10000
D_IN = 256
D_OUT = 256
NC, NS = 2, 16
DH = D_OUT // NC
E = 160000
CHUNK = 80
PER_TILE = E // NS
NCHUNK = PER_TILE // CHUNK
NP = 10240
ROW_T = NP // NS
CP = 80
LANES = 16
NBUF = 4


def _mm_body(x_ref, w_ref, o_ref):
    o_ref[...] = jnp.dot(x_ref[...], w_ref[...],
                         preferred_element_type=jnp.float32)


def _matmul_split(x, w):
    bm = 1000
    nm = N // bm
    return pl.pallas_call(
        _mm_body,
        grid=(NC, nm),
        in_specs=[
            pl.BlockSpec((bm, D_IN), lambda c, m: (m, 0)),
            pl.BlockSpec((D_IN, DH), lambda c, m: (0, c)),
        ],
        out_specs=pl.BlockSpec((bm, DH), lambda c, m: (c * nm + m, 0)),
        out_shape=jax.ShapeDtypeStruct((NC * N, DH), jnp.float32),
    )(x, w)


def _spmm_body(sup_hbm, ei_hbm, ew_hbm, out_hbm,
               sidx, didx, eww, rows, acc,
               sm0, sm1, sm2, sm3, sd0, sd1, sd2, sd3,
               sw0, sw1, sw2, sw3, sg0, sg1, sg2, sg3,
               ss0, ss1, ss2, ss3):
    sm = (sm0, sm1, sm2, sm3)
    sd = (sd0, sd1, sd2, sd3)
    sw = (sw0, sw1, sw2, sw3)
    sg = (sg0, sg1, sg2, sg3)
    ss = (ss0, ss1, ss2, ss3)
    c = lax.axis_index("c")
    s = lax.axis_index("s")
    coff = c * N

    @pl.loop(0, CP)
    def _zero(r):
        for k in range(DH // LANES):
            rows[0, r, pl.ds(k * LANES, LANES)] = jnp.zeros((LANES,),
                                                            jnp.float32)

    for j in range(ROW_T // CP):
        pltpu.sync_copy(rows.at[0],
                        acc.at[pl.ds(s * ROW_T + j * CP, CP)])
    plsc.subcore_barrier()

    def issue_meta(i, b):
        base = s * PER_TILE + i * CHUNK
        pltpu.async_copy(ei_hbm.at[pl.ds(base, CHUNK)], sidx.at[b], sm[b])
        pltpu.async_copy(ei_hbm.at[pl.ds(E + base, CHUNK)], didx.at[b],
                         sd[b])
        pltpu.async_copy(ew_hbm.at[pl.ds(base, CHUNK)], eww.at[b], sw[b])

    def wait_meta(b):
        pltpu.make_async_copy(ei_hbm.at[pl.ds(0, CHUNK)], sidx.at[b],
                              sm[b]).wait()
        pltpu.make_async_copy(ei_hbm.at[pl.ds(0, CHUNK)], didx.at[b],
                              sd[b]).wait()
        pltpu.make_async_copy(ew_hbm.at[pl.ds(0, CHUNK)], eww.at[b],
                              sw[b]).wait()

    def extract(b):
        for j in range(CHUNK // LANES):
            sl = pl.ds(j * LANES, LANES)
            sidx[b, sl] = sidx[b, sl] + coff

    def issue_gather(b):
        pltpu.async_copy(sup_hbm.at[sidx.at[b]], rows.at[b], sg[b])

    def wait_gather(b):
        pltpu.make_async_copy(sup_hbm.at[pl.ds(0, CHUNK)], rows.at[b],
                              sg[b]).wait()

    def scale(b):
        @pl.loop(0, CHUNK // LANES)
        def _sc(q):
            wv = eww[b, pl.ds(q * LANES, LANES)]
            for i in range(LANES):
                w = wv[i]
                e = q * LANES + i
                for k in range(DH // LANES):
                    sl = pl.ds(k * LANES, LANES)
                    rows[b, e, sl] = rows[b, e, sl] * w

    def issue_scatter(b):
        pltpu.async_copy(rows.at[b], acc.at[didx.at[b]], ss[b], add=True)

    def wait_scatter(b):
        pltpu.make_async_copy(sup_hbm.at[pl.ds(0, CHUNK)], rows.at[b],
                              ss[b]).wait()

    def step(i, ic, do_sw=True, do_nxt=True, do_meta=True):
        b = ic % NBUF
        bn = (ic + 1) % NBUF
        bn2 = (ic + 2) % NBUF
        if do_nxt:
            wait_meta(bn)
            extract(bn)
            issue_gather(bn)
        if do_meta:
            if do_sw:
                wait_scatter(bn2)
            issue_meta(i + 2, bn2)
        wait_gather(b)
        scale(b)
        issue_scatter(b)

    issue_meta(0, 0)
    issue_meta(1, 1)
    wait_meta(0)
    extract(0)
    issue_gather(0)
    step(0, 0, do_sw=False)
    step(1, 1, do_sw=False)
    step(2, 2)
    step(3, 3)

    @pl.loop(4, NCHUNK - 5, step=NBUF)
    def _main(i):
        for k in range(NBUF):
            step(i + k, k)

    step(NCHUNK - 5, 0)
    step(NCHUNK - 4, 1)
    step(NCHUNK - 3, 2)
    step(NCHUNK - 2, 3, do_meta=False)
    step(NCHUNK - 1, 0, do_nxt=False, do_meta=False)
    for b in range(NBUF):
        wait_scatter(b)

    plsc.subcore_barrier()

    nout = jnp.where(s == NS - 1, (N - (NS - 1) * ROW_T) // CP, ROW_T // CP)
    col0 = pl.multiple_of(c * DH, 128)

    @pl.loop(0, nout)
    def _out(j):
        r0 = pl.multiple_of(s * ROW_T + j * CP, 8)
        pltpu.sync_copy(acc.at[pl.ds(r0, CP)],
                        out_hbm.at[pl.ds(r0, CP), pl.ds(col0, DH)])


def _spmm_sc(sup, ei, ew):
    mesh = plsc.VectorSubcoreMesh(core_axis_name="c", subcore_axis_name="s",
                                  num_cores=NC, num_subcores=NS)
    run = pl.kernel(
        _spmm_body,
        out_type=jax.ShapeDtypeStruct((N, D_OUT), jnp.float32),
        mesh=mesh,
        scratch_types=[
            pltpu.VMEM((NBUF, CHUNK), jnp.int32),
            pltpu.VMEM((NBUF, CHUNK), jnp.int32),
            pltpu.VMEM((NBUF, CHUNK), jnp.float32),
            pltpu.VMEM((NBUF, CHUNK, DH), jnp.float32),
            pltpu.VMEM_SHARED((NP, DH), jnp.float32),
        ] + [pltpu.SemaphoreType.DMA] * 20,
    )
    return run(sup, ei, ew)


def kernel(input, edge_index, edge_weight, W):
    ei = edge_index.astype(jnp.int32).reshape(2 * E)
    sup = _matmul_split(input, W)
    return _spmm_sc(sup, ei, edge_weight)
